# baseline (speedup 1.0000x reference)
import jax
import jax.numpy as jnp
from jax import lax
from jax.experimental import pallas as pl
from jax.experimental.pallas import tpu as pltpu
from jax.experimental.pallas import tpu_sc as plsc

NUM_CORES = 2
NUM_SUBCORES = 16
LANES = 16
NUM_WORKERS = NUM_CORES * NUM_SUBCORES

BATCH = 16384
EMB_DIM = 32
B_PER_W = BATCH // NUM_WORKERS
IDX_CHUNK = 128
N_CHUNKS = B_PER_W // IDX_CHUNK
N_GROUPS = B_PER_W // LANES


def _bpr_kernel(users_hbm, item_i_hbm, item_j_hbm, user_emb_hbm,
                item_emb_hbm, w_hbm, b_hbm, out_pos_hbm, out_neg_hbm,
                uidx_v, iidx_v, jidx_v, u_rows, i_rows, j_rows,
                w_v, b_v, outp_v, outn_v, sem):
    wid = lax.axis_index("s") * NUM_CORES + lax.axis_index("c")
    base = wid * B_PER_W

    pltpu.sync_copy(users_hbm.at[pl.ds(base, B_PER_W)], uidx_v)
    pltpu.sync_copy(item_i_hbm.at[pl.ds(base, B_PER_W)], iidx_v)
    pltpu.sync_copy(item_j_hbm.at[pl.ds(base, B_PER_W)], jidx_v)
    pltpu.sync_copy(w_hbm, w_v)
    pltpu.sync_copy(b_hbm, b_v)

    copies = []
    for c in range(N_CHUNKS):
        sl = pl.ds(c * IDX_CHUNK, IDX_CHUNK)
        copies.append(pltpu.async_copy(
            user_emb_hbm.at[uidx_v.at[sl]], u_rows.at[sl], sem))
        copies.append(pltpu.async_copy(
            item_emb_hbm.at[iidx_v.at[sl]], i_rows.at[sl], sem))
        copies.append(pltpu.async_copy(
            item_emb_hbm.at[jidx_v.at[sl]], j_rows.at[sl], sem))
    for cp in copies:
        cp.wait()

    w0 = w_v[pl.ds(0, LANES)]
    w1 = w_v[pl.ds(LANES, LANES)]

    def prescale(n, carry):
        u_rows[n, pl.ds(0, LANES)] = u_rows[n, pl.ds(0, LANES)] * w0
        u_rows[n, pl.ds(LANES, LANES)] = u_rows[n, pl.ds(LANES, LANES)] * w1
        return carry

    lax.fori_loop(0, B_PER_W, prescale, 0)

    bias = b_v[pl.ds(0, LANES)]
    iota16 = lax.iota(jnp.int32, LANES)

    def group_body(g, carry):
        rvec = iota16 + g * LANES
        acc_p = bias
        acc_n = bias
        for d in range(EMB_DIM):
            cvec = jnp.full((LANES,), d, jnp.int32)
            ud = plsc.load_gather(u_rows, [rvec, cvec])
            iv = plsc.load_gather(i_rows, [rvec, cvec])
            jv = plsc.load_gather(j_rows, [rvec, cvec])
            acc_p = acc_p + ud * iv
            acc_n = acc_n + ud * jv
        outp_v[pl.ds(g * LANES, LANES)] = acc_p
        outn_v[pl.ds(g * LANES, LANES)] = acc_n
        return carry

    lax.fori_loop(0, N_GROUPS, group_body, 0)

    pltpu.sync_copy(outp_v, out_pos_hbm.at[pl.ds(base, B_PER_W)])
    pltpu.sync_copy(outn_v, out_neg_hbm.at[pl.ds(base, B_PER_W)])


@jax.jit
def kernel(users, item_i, item_j, user_emb, item_emb, W, b):
    mesh = plsc.VectorSubcoreMesh(core_axis_name="c", subcore_axis_name="s")
    w_flat = W.reshape(EMB_DIM).astype(jnp.float32)
    b_vec = jnp.broadcast_to(b.reshape(1), (LANES,)).astype(jnp.float32)

    run = pl.kernel(
        _bpr_kernel,
        out_type=(
            jax.ShapeDtypeStruct((BATCH,), jnp.float32),
            jax.ShapeDtypeStruct((BATCH,), jnp.float32),
        ),
        mesh=mesh,
        compiler_params=pltpu.CompilerParams(needs_layout_passes=False,
                                             use_tc_tiling_on_sc=False),
        scratch_types=[
            pltpu.VMEM((B_PER_W,), jnp.int32),
            pltpu.VMEM((B_PER_W,), jnp.int32),
            pltpu.VMEM((B_PER_W,), jnp.int32),
            pltpu.VMEM((B_PER_W, EMB_DIM), jnp.float32),
            pltpu.VMEM((B_PER_W, EMB_DIM), jnp.float32),
            pltpu.VMEM((B_PER_W, EMB_DIM), jnp.float32),
            pltpu.VMEM((EMB_DIM,), jnp.float32),
            pltpu.VMEM((LANES,), jnp.float32),
            pltpu.VMEM((B_PER_W,), jnp.float32),
            pltpu.VMEM((B_PER_W,), jnp.float32),
            pltpu.SemaphoreType.DMA,
        ],
        name="bpr_sc",
    )
    out_pos, out_neg = run(
        users.astype(jnp.int32), item_i.astype(jnp.int32),
        item_j.astype(jnp.int32), user_emb, item_emb, w_flat, b_vec)
    return out_pos, out_neg
